# Initial kernel scaffold; baseline (speedup 1.0000x reference)
import jax
import jax.numpy as jnp
from jax.experimental import pallas as pl


def kernel(predicted_locs, predicted_logits, boxes, labels, priors_cxcy):
    raise NotImplementedError("write your pallas kernel here")


# speedup vs baseline: 3.3460x; 3.3460x over previous
import functools

import jax
import jax.numpy as jnp
from jax.experimental import pallas as pl

_THRESHOLD = 0.5
_NEG_POS_RATIO = 3
_ALPHA = 1.0
_BISECT_ITERS = 40


def _match_kernel(boxes_ref, labels_ref, pxy_ref, pcxcy_ref, plocs_ref,
                  tc_ref, locsum_ref, npos_ref):
    P = pxy_ref.shape[1]
    O = boxes_ref.shape[1]

    bx = boxes_ref[0]
    lab_col = labels_ref[0]
    pxy = pxy_ref[...]
    pcx = pcxcy_ref[...]
    plocs = plocs_ref[0]

    bx0 = bx[:, 0:1]
    by0 = bx[:, 1:2]
    bx1 = bx[:, 2:3]
    by1 = bx[:, 3:4]

    px0 = pxy[0:1, :]
    py0 = pxy[1:2, :]
    px1 = pxy[2:3, :]
    py1 = pxy[3:4, :]

    lt_x = jnp.maximum(bx0, px0)
    lt_y = jnp.maximum(by0, py0)
    rb_x = jnp.minimum(bx1, px1)
    rb_y = jnp.minimum(by1, py1)
    iw = jnp.maximum(rb_x - lt_x, 0.0)
    ih = jnp.maximum(rb_y - lt_y, 0.0)
    inter = iw * ih
    area_b = (bx1 - bx0) * (by1 - by0)
    area_p = (px1 - px0) * (py1 - py0)
    overlap = inter / (area_b + area_p - inter)

    iota_o = jax.lax.broadcasted_iota(jnp.int32, (O, P), 0)
    iota_p = jax.lax.broadcasted_iota(jnp.int32, (O, P), 1)

    ovr = jnp.max(overlap, axis=0, keepdims=True)
    obj = jnp.min(jnp.where(overlap == ovr, iota_o, O),
                  axis=0, keepdims=True)

    rmax = jnp.max(overlap, axis=1, keepdims=True)
    pbest = jnp.min(jnp.where(overlap == rmax, iota_p, P),
                    axis=1, keepdims=True)

    hit = iota_p == pbest
    oass = jnp.max(jnp.where(hit, iota_o, -1), axis=0, keepdims=True)
    forced = oass >= 0
    obj = jnp.where(forced, oass, obj)
    ovr = jnp.where(forced, 1.0, ovr)

    pick = iota_o == obj

    lab = jnp.sum(jnp.where(pick, lab_col, 0), axis=0, keepdims=True)
    lab = jnp.where(ovr < _THRESHOLD, 0, lab)
    tc_ref[0] = lab

    pos = (lab != 0).astype(jnp.float32)
    npos_ref[0] = jnp.sum((lab != 0).astype(jnp.int32), axis=1,
                          keepdims=True)

    bcx = (bx0 + bx1) * 0.5
    bcy = (by0 + by1) * 0.5
    bw = bx1 - bx0
    bh = by1 - by0
    g_cx = jnp.sum(jnp.where(pick, bcx, 0.0), axis=0, keepdims=True)
    g_cy = jnp.sum(jnp.where(pick, bcy, 0.0), axis=0, keepdims=True)
    g_w = jnp.sum(jnp.where(pick, bw, 0.0), axis=0, keepdims=True)
    g_h = jnp.sum(jnp.where(pick, bh, 0.0), axis=0, keepdims=True)

    p_cx = pcx[0:1, :]
    p_cy = pcx[1:2, :]
    p_w = pcx[2:3, :]
    p_h = pcx[3:4, :]

    t0 = (g_cx - p_cx) / (p_w * 0.1)
    t1 = (g_cy - p_cy) / (p_h * 0.1)
    t2 = jnp.log(g_w / p_w) * 5.0
    t3 = jnp.log(g_h / p_h) * 5.0

    d = (jnp.abs(plocs[0:1, :] - t0) + jnp.abs(plocs[1:2, :] - t1)
         + jnp.abs(plocs[2:3, :] - t2) + jnp.abs(plocs[3:4, :] - t3))
    locsum_ref[0] = jnp.sum(d * pos, axis=1, keepdims=True)


def _conf_kernel(logits_ref, tc_ref, npos_ref, cepos_ref, hard_ref):
    P, C = logits_ref.shape[1], logits_ref.shape[2]

    x = logits_ref[0]
    tc = tc_ref[0]

    iota_c = jax.lax.broadcasted_iota(jnp.int32, (P, C), 1)
    onehot = (iota_c == tc).astype(jnp.float32)
    true_logit = jnp.sum(x * onehot, axis=1, keepdims=True)

    m = jnp.max(x, axis=1, keepdims=True)
    logz = m + jnp.log(jnp.sum(jnp.exp(x - m), axis=1, keepdims=True))
    ce = logz - true_logit
    ce = jnp.where(jnp.isnan(ce) | jnp.isinf(ce), 0.0, ce)

    pos = tc != 0
    posf = pos.astype(jnp.float32)
    cepos_ref[0] = jnp.sum(ce * posf, axis=0, keepdims=True)

    ce_neg = jnp.where(pos, 0.0, ce)

    n = npos_ref[0, 0, 0]
    k = jnp.minimum(_NEG_POS_RATIO * n, P)

    hi0 = jnp.max(ce_neg)
    lo0 = jnp.float32(-1.0)

    def body(_, carry):
        lo, hi = carry
        mid = (lo + hi) * 0.5
        cnt = jnp.sum(ce_neg > mid, dtype=jnp.int32)
        take_hi = cnt >= k
        return (jnp.where(take_hi, mid, lo), jnp.where(take_hi, hi, mid))

    _, t = jax.lax.fori_loop(0, _BISECT_ITERS, body, (lo0, hi0))

    gt = ce_neg > t
    cnt_gt = jnp.sum(gt.astype(jnp.float32), axis=0, keepdims=True)
    sum_gt = jnp.sum(jnp.where(gt, ce_neg, 0.0), axis=0, keepdims=True)
    hard = sum_gt + t * (k.astype(jnp.float32) - cnt_gt)
    hard_ref[0] = jnp.where(k > 0, hard, 0.0)


def kernel(predicted_locs, predicted_logits, boxes, labels, priors_cxcy):
    B, P, C = predicted_logits.shape
    O = boxes.shape[1]

    priors_xy_t = jnp.concatenate(
        [priors_cxcy[:, :2] - priors_cxcy[:, 2:] * 0.5,
         priors_cxcy[:, :2] + priors_cxcy[:, 2:] * 0.5], axis=-1).T
    priors_cxcy_t = priors_cxcy.T
    plocs_t = jnp.swapaxes(predicted_locs, 1, 2)
    labels_c = labels[..., None].astype(jnp.int32)

    tc, locsum, npos = pl.pallas_call(
        _match_kernel,
        grid=(B,),
        in_specs=[
            pl.BlockSpec((1, O, 4), lambda b: (b, 0, 0)),
            pl.BlockSpec((1, O, 1), lambda b: (b, 0, 0)),
            pl.BlockSpec((4, P), lambda b: (0, 0)),
            pl.BlockSpec((4, P), lambda b: (0, 0)),
            pl.BlockSpec((1, 4, P), lambda b: (b, 0, 0)),
        ],
        out_specs=[
            pl.BlockSpec((1, 1, P), lambda b: (b, 0, 0)),
            pl.BlockSpec((1, 1, 1), lambda b: (b, 0, 0)),
            pl.BlockSpec((1, 1, 1), lambda b: (b, 0, 0)),
        ],
        out_shape=[
            jax.ShapeDtypeStruct((B, 1, P), jnp.int32),
            jax.ShapeDtypeStruct((B, 1, 1), jnp.float32),
            jax.ShapeDtypeStruct((B, 1, 1), jnp.int32),
        ],
    )(boxes, labels_c, priors_xy_t, priors_cxcy_t, plocs_t)

    tc_c = jnp.swapaxes(tc, 1, 2)

    cepos, hard = pl.pallas_call(
        _conf_kernel,
        grid=(B,),
        in_specs=[
            pl.BlockSpec((1, P, C), lambda b: (b, 0, 0)),
            pl.BlockSpec((1, P, 1), lambda b: (b, 0, 0)),
            pl.BlockSpec((1, 1, 1), lambda b: (b, 0, 0)),
        ],
        out_specs=[
            pl.BlockSpec((1, 1, 1), lambda b: (b, 0, 0)),
            pl.BlockSpec((1, 1, 1), lambda b: (b, 0, 0)),
        ],
        out_shape=[
            jax.ShapeDtypeStruct((B, 1, 1), jnp.float32),
            jax.ShapeDtypeStruct((B, 1, 1), jnp.float32),
        ],
    )(predicted_logits, tc_c, npos)

    n_total = jnp.sum(npos)
    denom = jnp.maximum(n_total, 1).astype(jnp.float32)

    loc_loss = jnp.sum(locsum) / jnp.maximum(n_total * 4, 1).astype(jnp.float32)
    loc_loss = jnp.where(jnp.isnan(loc_loss) | jnp.isinf(loc_loss), 0.0,
                         loc_loss)
    conf_loss = (jnp.sum(hard) + jnp.sum(cepos)) / denom
    return conf_loss + _ALPHA * loc_loss


# speedup vs baseline: 11.7972x; 3.5258x over previous
import functools

import jax
import jax.numpy as jnp
from jax.experimental import pallas as pl
from jax.experimental.pallas import tpu as pltpu

_THRESHOLD = 0.5
_NEG_POS_RATIO = 3
_ALPHA = 1.0
_BISECT_ITERS = 40


def _match_kernel(boxes_ref, labels_ref, pxy_ref, pcxcy_ref, plocs_ref,
                  tc_ref, locsum_ref, npos_ref):
    P = pxy_ref.shape[1]
    O = boxes_ref.shape[1]

    bx = boxes_ref[0]
    lab_col = labels_ref[0]
    pxy = pxy_ref[...]
    pcx = pcxcy_ref[...]
    plocs = plocs_ref[0]

    bx0 = bx[:, 0:1]
    by0 = bx[:, 1:2]
    bx1 = bx[:, 2:3]
    by1 = bx[:, 3:4]

    px0 = pxy[0:1, :]
    py0 = pxy[1:2, :]
    px1 = pxy[2:3, :]
    py1 = pxy[3:4, :]

    lt_x = jnp.maximum(bx0, px0)
    lt_y = jnp.maximum(by0, py0)
    rb_x = jnp.minimum(bx1, px1)
    rb_y = jnp.minimum(by1, py1)
    iw = jnp.maximum(rb_x - lt_x, 0.0)
    ih = jnp.maximum(rb_y - lt_y, 0.0)
    inter = iw * ih
    area_b = (bx1 - bx0) * (by1 - by0)
    area_p = (px1 - px0) * (py1 - py0)
    overlap = inter / (area_b + area_p - inter)

    iota_o = jax.lax.broadcasted_iota(jnp.int32, (O, P), 0)
    iota_p = jax.lax.broadcasted_iota(jnp.int32, (O, P), 1)

    ovr = jnp.max(overlap, axis=0, keepdims=True)
    obj = jnp.min(jnp.where(overlap == ovr, iota_o, O),
                  axis=0, keepdims=True)

    rmax = jnp.max(overlap, axis=1, keepdims=True)
    pbest = jnp.min(jnp.where(overlap == rmax, iota_p, P),
                    axis=1, keepdims=True)

    hit = iota_p == pbest
    oass = jnp.max(jnp.where(hit, iota_o, -1), axis=0, keepdims=True)
    forced = oass >= 0
    obj = jnp.where(forced, oass, obj)
    ovr = jnp.where(forced, 1.0, ovr)

    pick = iota_o == obj

    lab = jnp.sum(jnp.where(pick, lab_col, 0), axis=0, keepdims=True)
    lab = jnp.where(ovr < _THRESHOLD, 0, lab)
    tc_ref[0] = lab

    pos = (lab != 0).astype(jnp.float32)
    npos_ref[0] = jnp.sum((lab != 0).astype(jnp.int32), axis=1,
                          keepdims=True)

    bcx = (bx0 + bx1) * 0.5
    bcy = (by0 + by1) * 0.5
    bw = bx1 - bx0
    bh = by1 - by0
    g_cx = jnp.sum(jnp.where(pick, bcx, 0.0), axis=0, keepdims=True)
    g_cy = jnp.sum(jnp.where(pick, bcy, 0.0), axis=0, keepdims=True)
    g_w = jnp.sum(jnp.where(pick, bw, 0.0), axis=0, keepdims=True)
    g_h = jnp.sum(jnp.where(pick, bh, 0.0), axis=0, keepdims=True)

    p_cx = pcx[0:1, :]
    p_cy = pcx[1:2, :]
    p_w = pcx[2:3, :]
    p_h = pcx[3:4, :]

    t0 = (g_cx - p_cx) / (p_w * 0.1)
    t1 = (g_cy - p_cy) / (p_h * 0.1)
    t2 = jnp.log(g_w / p_w) * 5.0
    t3 = jnp.log(g_h / p_h) * 5.0

    d = (jnp.abs(plocs[0:1, :] - t0) + jnp.abs(plocs[1:2, :] - t1)
         + jnp.abs(plocs[2:3, :] - t2) + jnp.abs(plocs[3:4, :] - t3))
    locsum_ref[0] = jnp.sum(d * pos, axis=1, keepdims=True)


def _ce_kernel(logits_ref, tc_ref, cepos_ref, ceneg_ref):
    P, C = logits_ref.shape[1], logits_ref.shape[2]

    x = logits_ref[0]
    tc = tc_ref[0]

    iota_c = jax.lax.broadcasted_iota(jnp.int32, (P, C), 1)
    onehot = (iota_c == tc).astype(jnp.float32)
    true_logit = jnp.sum(x * onehot, axis=1, keepdims=True)

    m = jnp.max(x, axis=1, keepdims=True)
    logz = m + jnp.log(jnp.sum(jnp.exp(x - m), axis=1, keepdims=True))
    ce = logz - true_logit
    ce = jnp.where(jnp.isnan(ce) | jnp.isinf(ce), 0.0, ce)

    pos = tc != 0
    posf = pos.astype(jnp.float32)
    cepos_ref[0] = jnp.sum(ce * posf, axis=0, keepdims=True)
    ceneg_ref[0] = jnp.where(pos, 0.0, ce)


def _mine_kernel(ceneg_ref, npos_ref, hard_ref):
    B, P = ceneg_ref.shape
    x = ceneg_ref[...]
    n = npos_ref[...]
    kf = jnp.minimum(_NEG_POS_RATIO * n, P).astype(jnp.float32)

    hi0 = jnp.max(x, axis=1, keepdims=True)
    lo0 = jnp.full((B, 1), -1.0, jnp.float32)

    def body(_, carry):
        lo, hi = carry
        mid = (lo + hi) * 0.5
        cnt = jnp.sum((x > mid).astype(jnp.float32), axis=1, keepdims=True)
        take_hi = cnt >= kf
        return (jnp.where(take_hi, mid, lo), jnp.where(take_hi, hi, mid))

    _, t = jax.lax.fori_loop(0, _BISECT_ITERS, body, (lo0, hi0))

    gt = x > t
    cnt_gt = jnp.sum(gt.astype(jnp.float32), axis=1, keepdims=True)
    sum_gt = jnp.sum(jnp.where(gt, x, 0.0), axis=1, keepdims=True)
    hard = sum_gt + t * (kf - cnt_gt)
    hard_ref[...] = jnp.where(kf > 0.0, hard, 0.0)


def kernel(predicted_locs, predicted_logits, boxes, labels, priors_cxcy):
    B, P, C = predicted_logits.shape
    O = boxes.shape[1]

    priors_xy_t = jnp.concatenate(
        [priors_cxcy[:, :2] - priors_cxcy[:, 2:] * 0.5,
         priors_cxcy[:, :2] + priors_cxcy[:, 2:] * 0.5], axis=-1).T
    priors_cxcy_t = priors_cxcy.T
    plocs_t = jnp.swapaxes(predicted_locs, 1, 2)
    labels_c = labels[..., None].astype(jnp.int32)

    tc, locsum, npos = pl.pallas_call(
        _match_kernel,
        grid=(B,),
        in_specs=[
            pl.BlockSpec((1, O, 4), lambda b: (b, 0, 0)),
            pl.BlockSpec((1, O, 1), lambda b: (b, 0, 0)),
            pl.BlockSpec((4, P), lambda b: (0, 0)),
            pl.BlockSpec((4, P), lambda b: (0, 0)),
            pl.BlockSpec((1, 4, P), lambda b: (b, 0, 0)),
        ],
        out_specs=[
            pl.BlockSpec((1, 1, P), lambda b: (b, 0, 0)),
            pl.BlockSpec((1, 1, 1), lambda b: (b, 0, 0)),
            pl.BlockSpec((1, 1, 1), lambda b: (b, 0, 0)),
        ],
        out_shape=[
            jax.ShapeDtypeStruct((B, 1, P), jnp.int32),
            jax.ShapeDtypeStruct((B, 1, 1), jnp.float32),
            jax.ShapeDtypeStruct((B, 1, 1), jnp.int32),
        ],
        compiler_params=pltpu.CompilerParams(
            dimension_semantics=("parallel",)),
    )(boxes, labels_c, priors_xy_t, priors_cxcy_t, plocs_t)

    tc_c = jnp.swapaxes(tc, 1, 2)

    cepos, ceneg = pl.pallas_call(
        _ce_kernel,
        grid=(B,),
        in_specs=[
            pl.BlockSpec((1, P, C), lambda b: (b, 0, 0)),
            pl.BlockSpec((1, P, 1), lambda b: (b, 0, 0)),
        ],
        out_specs=[
            pl.BlockSpec((1, 1, 1), lambda b: (b, 0, 0)),
            pl.BlockSpec((1, P, 1), lambda b: (b, 0, 0)),
        ],
        out_shape=[
            jax.ShapeDtypeStruct((B, 1, 1), jnp.float32),
            jax.ShapeDtypeStruct((B, P, 1), jnp.float32),
        ],
        compiler_params=pltpu.CompilerParams(
            dimension_semantics=("parallel",)),
    )(predicted_logits, tc_c)

    hard = pl.pallas_call(
        _mine_kernel,
        in_specs=[
            pl.BlockSpec((B, P), lambda: (0, 0)),
            pl.BlockSpec((B, 1), lambda: (0, 0)),
        ],
        out_specs=pl.BlockSpec((B, 1), lambda: (0, 0)),
        out_shape=jax.ShapeDtypeStruct((B, 1), jnp.float32),
    )(ceneg[..., 0], npos[..., 0])

    n_total = jnp.sum(npos)
    denom = jnp.maximum(n_total, 1).astype(jnp.float32)

    loc_loss = jnp.sum(locsum) / jnp.maximum(n_total * 4, 1).astype(jnp.float32)
    loc_loss = jnp.where(jnp.isnan(loc_loss) | jnp.isinf(loc_loss), 0.0,
                         loc_loss)
    conf_loss = (jnp.sum(hard) + jnp.sum(cepos)) / denom
    return conf_loss + _ALPHA * loc_loss
